# initial kernel scaffold (unmeasured)
import jax
import jax.numpy as jnp
from jax import lax
from jax.experimental import pallas as pl
from jax.experimental.pallas import tpu as pltpu


def kernel(
    x,
):
    def body(*refs):
        pass

    out_shape = jax.ShapeDtypeStruct(..., jnp.float32)
    return pl.pallas_call(body, out_shape=out_shape)(...)


# baseline (device time: 104342 ns/iter reference)
import jax
import jax.numpy as jnp
from jax import lax
from jax.experimental import pallas as pl
from jax.experimental.pallas import tpu as pltpu

M = 1024
N_LOC = 8192
K_OUT = 32
N_Y = 4

CHUNK = 512
N_CHUNK = N_LOC // CHUNK
K_CHUNK = 11
N_CAND = N_CHUNK * K_CHUNK

ROW_BLK = 256

_NEG = float("-inf")


def _local_cand_body(x_ref, cand_ref):
    for c in range(N_CHUNK):
        prev = None
        for k in range(K_CHUNK):
            xs = x_ref[:, c * CHUNK:(c + 1) * CHUNK]
            cur = xs if prev is None else jnp.where(xs < prev, xs, _NEG)
            m = jnp.max(cur, axis=1, keepdims=True)
            j = c * K_CHUNK + k
            cand_ref[:, j:j + 1] = m
            prev = m


def _merge_body(cand_ref, out_ref, sbuf_ref, gath_ref, send_sems, recv_sems):
    my_x = lax.axis_index("x")
    my_y = lax.axis_index("y")
    my_z = lax.axis_index("z")

    prev = None
    for i in range(K_OUT):
        cs = cand_ref[:, :]
        cur = cs if prev is None else jnp.where(cs < prev, cs, _NEG)
        m = jnp.max(cur, axis=1, keepdims=True)
        sbuf_ref[:, i:i + 1] = m
        prev = m
    gath_ref[0, :, :] = sbuf_ref[:, :]

    barrier = pltpu.get_barrier_semaphore()
    for d in (1, 2, 3):
        peer_y = lax.rem(my_y + d, N_Y)
        pl.semaphore_signal(
            barrier, inc=1,
            device_id=(my_x, peer_y, my_z),
            device_id_type=pl.DeviceIdType.MESH,
        )
    pl.semaphore_wait(barrier, 3)

    rdmas = []
    for d in (1, 2, 3):
        peer_y = lax.rem(my_y + d, N_Y)
        rdma = pltpu.make_async_remote_copy(
            src_ref=sbuf_ref,
            dst_ref=gath_ref.at[N_Y - d],
            send_sem=send_sems.at[d - 1],
            recv_sem=recv_sems.at[d - 1],
            device_id=(my_x, peer_y, my_z),
            device_id_type=pl.DeviceIdType.MESH,
        )
        rdma.start()
        rdmas.append(rdma)
    for r in rdmas:
        r.wait_recv()

    prev = None
    for i in range(K_OUT):
        g = gath_ref[:, :, :]
        cur = g if prev is None else jnp.where(g < prev, g, _NEG)
        m = jnp.max(jnp.max(cur, axis=2), axis=0)
        out_ref[:, i:i + 1] = m[:, None]
        prev = m[None, :, None]

    for r in rdmas:
        r.wait_send()


def kernel(x):
    cand = pl.pallas_call(
        _local_cand_body,
        grid=(M // ROW_BLK,),
        in_specs=[
            pl.BlockSpec((ROW_BLK, N_LOC), lambda i: (i, 0),
                         memory_space=pltpu.VMEM),
        ],
        out_specs=pl.BlockSpec((ROW_BLK, N_CAND), lambda i: (i, 0),
                               memory_space=pltpu.VMEM),
        out_shape=jax.ShapeDtypeStruct((M, N_CAND), jnp.float32),
    )(x)

    return pl.pallas_call(
        _merge_body,
        in_specs=[pl.BlockSpec(memory_space=pltpu.VMEM)],
        out_specs=pl.BlockSpec(memory_space=pltpu.VMEM),
        out_shape=jax.ShapeDtypeStruct((M, K_OUT), jnp.float32),
        scratch_shapes=[
            pltpu.VMEM((M, K_OUT), jnp.float32),
            pltpu.VMEM((N_Y, M, K_OUT), jnp.float32),
            pltpu.SemaphoreType.DMA((N_Y - 1,)),
            pltpu.SemaphoreType.DMA((N_Y - 1,)),
        ],
        compiler_params=pltpu.CompilerParams(collective_id=0),
    )(cand)


# device time: 71468 ns/iter; 1.4600x vs baseline; 1.4600x over previous
import jax
import jax.numpy as jnp
from jax import lax
from jax.experimental import pallas as pl
from jax.experimental.pallas import tpu as pltpu

M = 1024
N_LOC = 8192
K_OUT = 32
N_Y = 4

LANES = 128
N_SLICE = N_LOC // LANES
K_LANE = 6

ROW_BLK = 256

_NEG = float("-inf")


def _local_topk_body(x_ref, out_ref):
    neg = jnp.full((ROW_BLK, LANES), _NEG, jnp.float32)
    t = [neg] * K_LANE
    for c in range(N_SLICE):
        s = x_ref[:, c * LANES:(c + 1) * LANES]
        for j in range(K_LANE):
            hi = jnp.maximum(t[j], s)
            s = jnp.minimum(t[j], s)
            t[j] = hi

    prev = None
    for i in range(K_OUT):
        if prev is None:
            best = t[0]
        else:
            best = jnp.full((ROW_BLK, LANES), _NEG, jnp.float32)
            for j in reversed(range(K_LANE)):
                best = jnp.where(t[j] < prev, t[j], best)
        m = jnp.max(best, axis=1, keepdims=True)
        out_ref[:, i:i + 1] = m
        prev = m


def _merge_body(cand_ref, out_ref, gath_ref, send_sems, recv_sems):
    my_x = lax.axis_index("x")
    my_y = lax.axis_index("y")
    my_z = lax.axis_index("z")

    gath_ref[0, :, :] = cand_ref[:, :]

    barrier = pltpu.get_barrier_semaphore()
    for d in (1, 2, 3):
        peer_y = lax.rem(my_y + d, N_Y)
        pl.semaphore_signal(
            barrier, inc=1,
            device_id=(my_x, peer_y, my_z),
            device_id_type=pl.DeviceIdType.MESH,
        )
    pl.semaphore_wait(barrier, 3)

    rdmas = []
    for d in (1, 2, 3):
        peer_y = lax.rem(my_y + d, N_Y)
        rdma = pltpu.make_async_remote_copy(
            src_ref=cand_ref,
            dst_ref=gath_ref.at[N_Y - d],
            send_sem=send_sems.at[d - 1],
            recv_sem=recv_sems.at[d - 1],
            device_id=(my_x, peer_y, my_z),
            device_id_type=pl.DeviceIdType.MESH,
        )
        rdma.start()
        rdmas.append(rdma)
    for r in rdmas:
        r.wait_recv()

    prev = None
    for i in range(K_OUT):
        g = gath_ref[:, :, :]
        cur = g if prev is None else jnp.where(g < prev, g, _NEG)
        m = jnp.max(jnp.max(cur, axis=2), axis=0)
        out_ref[:, i:i + 1] = m[:, None]
        prev = m[None, :, None]

    for r in rdmas:
        r.wait_send()


def kernel(x):
    cand = pl.pallas_call(
        _local_topk_body,
        grid=(M // ROW_BLK,),
        in_specs=[
            pl.BlockSpec((ROW_BLK, N_LOC), lambda i: (i, 0),
                         memory_space=pltpu.VMEM),
        ],
        out_specs=pl.BlockSpec((ROW_BLK, K_OUT), lambda i: (i, 0),
                               memory_space=pltpu.VMEM),
        out_shape=jax.ShapeDtypeStruct((M, K_OUT), jnp.float32),
    )(x)

    return pl.pallas_call(
        _merge_body,
        in_specs=[pl.BlockSpec(memory_space=pltpu.VMEM)],
        out_specs=pl.BlockSpec(memory_space=pltpu.VMEM),
        out_shape=jax.ShapeDtypeStruct((M, K_OUT), jnp.float32),
        scratch_shapes=[
            pltpu.VMEM((N_Y, M, K_OUT), jnp.float32),
            pltpu.SemaphoreType.DMA((N_Y - 1,)),
            pltpu.SemaphoreType.DMA((N_Y - 1,)),
        ],
        compiler_params=pltpu.CompilerParams(collective_id=0),
    )(cand)


# device time: 56306 ns/iter; 1.8531x vs baseline; 1.2693x over previous
import jax
import jax.numpy as jnp
from jax import lax
from jax.experimental import pallas as pl
from jax.experimental.pallas import tpu as pltpu

M = 1024
N_LOC = 8192
K_OUT = 32
N_Y = 4

LANES = 128
N_SLICE = N_LOC // LANES
K_LANE = 6

ROW_BLK = 256

_NEG = float("-inf")


def _local_topk_body(x_ref, out_ref):
    neg = jnp.full((ROW_BLK, LANES), _NEG, jnp.float32)
    t = [neg] * K_LANE
    for c in range(N_SLICE):
        s = x_ref[:, c * LANES:(c + 1) * LANES]
        for j in range(K_LANE):
            hi = jnp.maximum(t[j], s)
            s = jnp.minimum(t[j], s)
            t[j] = hi

    prev = None
    for i in range(K_OUT):
        if prev is None:
            best = t[0]
        else:
            best = jnp.full((ROW_BLK, LANES), _NEG, jnp.float32)
            for j in reversed(range(K_LANE)):
                best = jnp.where(t[j] < prev, t[j], best)
        m = jnp.max(best, axis=1, keepdims=True)
        out_ref[:, i:i + 1] = m
        prev = m


def _merge_body(cand_ref, out_ref, gath_ref, outT_ref, send_sems, recv_sems):
    my_x = lax.axis_index("x")
    my_y = lax.axis_index("y")
    my_z = lax.axis_index("z")

    gath_ref[0, :, :] = cand_ref[:, :]

    barrier = pltpu.get_barrier_semaphore()
    for d in (1, 2, 3):
        peer_y = lax.rem(my_y + d, N_Y)
        pl.semaphore_signal(
            barrier, inc=1,
            device_id=(my_x, peer_y, my_z),
            device_id_type=pl.DeviceIdType.MESH,
        )
    pl.semaphore_wait(barrier, 3)

    rdmas = []
    for d in (1, 2, 3):
        peer_y = lax.rem(my_y + d, N_Y)
        rdma = pltpu.make_async_remote_copy(
            src_ref=cand_ref,
            dst_ref=gath_ref.at[N_Y - d],
            send_sem=send_sems.at[d - 1],
            recv_sem=recv_sems.at[d - 1],
            device_id=(my_x, peer_y, my_z),
            device_id_type=pl.DeviceIdType.MESH,
        )
        rdma.start()
        rdmas.append(rdma)
    for r in rdmas:
        r.wait_recv()

    g2 = jnp.concatenate(
        [gath_ref[0], gath_ref[1], gath_ref[2], gath_ref[3]], axis=1)
    g2t = g2.T
    prev = None
    for i in range(K_OUT):
        cur = g2t if prev is None else jnp.where(g2t < prev, g2t, _NEG)
        m = jnp.max(cur, axis=0, keepdims=True)
        outT_ref[i:i + 1, :] = m
        prev = m
    out_ref[:, :] = outT_ref[:, :].T

    for r in rdmas:
        r.wait_send()


def kernel(x):
    cand = pl.pallas_call(
        _local_topk_body,
        grid=(M // ROW_BLK,),
        in_specs=[
            pl.BlockSpec((ROW_BLK, N_LOC), lambda i: (i, 0),
                         memory_space=pltpu.VMEM),
        ],
        out_specs=pl.BlockSpec((ROW_BLK, K_OUT), lambda i: (i, 0),
                               memory_space=pltpu.VMEM),
        out_shape=jax.ShapeDtypeStruct((M, K_OUT), jnp.float32),
    )(x)

    return pl.pallas_call(
        _merge_body,
        in_specs=[pl.BlockSpec(memory_space=pltpu.VMEM)],
        out_specs=pl.BlockSpec(memory_space=pltpu.VMEM),
        out_shape=jax.ShapeDtypeStruct((M, K_OUT), jnp.float32),
        scratch_shapes=[
            pltpu.VMEM((N_Y, M, K_OUT), jnp.float32),
            pltpu.VMEM((K_OUT, M), jnp.float32),
            pltpu.SemaphoreType.DMA((N_Y - 1,)),
            pltpu.SemaphoreType.DMA((N_Y - 1,)),
        ],
        compiler_params=pltpu.CompilerParams(collective_id=0),
    )(cand)


# device time: 39731 ns/iter; 2.6262x vs baseline; 1.4172x over previous
import jax
import jax.numpy as jnp
from jax import lax
from jax.experimental import pallas as pl
from jax.experimental.pallas import tpu as pltpu

M = 1024
N_LOC = 8192
K_OUT = 32
N_Y = 4

LANES = 128
N_SLICE = N_LOC // LANES
K_LANE = 6

ROW_BLK = 256

_NEG = float("-inf")


def _local_topk_body(x_ref, out_ref):
    neg = jnp.full((ROW_BLK, LANES), _NEG, jnp.float32)
    t = [neg] * K_LANE
    for c in range(N_SLICE):
        s = x_ref[:, c * LANES:(c + 1) * LANES]
        for j in range(K_LANE):
            hi = jnp.maximum(t[j], s)
            s = jnp.minimum(t[j], s)
            t[j] = hi

    prev = None
    for i in range(K_OUT):
        if prev is None:
            best = t[0]
        else:
            best = jnp.full((ROW_BLK, LANES), _NEG, jnp.float32)
            for j in reversed(range(K_LANE)):
                best = jnp.where(t[j] < prev, t[j], best)
        m = jnp.max(best, axis=1, keepdims=True)
        out_ref[:, i:i + 1] = m
        prev = m


def _merge_body(cand_ref, out_ref, candT_ref, gathT_ref, outT_ref,
                send_sems, recv_sems):
    my_x = lax.axis_index("x")
    my_y = lax.axis_index("y")
    my_z = lax.axis_index("z")

    candT_ref[:, :] = cand_ref[:, :].T
    gathT_ref[0:K_OUT, :] = candT_ref[:, :]

    barrier = pltpu.get_barrier_semaphore()
    for d in (1, 2, 3):
        peer_y = lax.rem(my_y + d, N_Y)
        pl.semaphore_signal(
            barrier, inc=1,
            device_id=(my_x, peer_y, my_z),
            device_id_type=pl.DeviceIdType.MESH,
        )
    pl.semaphore_wait(barrier, 3)

    rdmas = []
    for d in (1, 2, 3):
        peer_y = lax.rem(my_y + d, N_Y)
        rdma = pltpu.make_async_remote_copy(
            src_ref=candT_ref,
            dst_ref=gathT_ref.at[pl.ds((N_Y - d) * K_OUT, K_OUT)],
            send_sem=send_sems.at[d - 1],
            recv_sem=recv_sems.at[d - 1],
            device_id=(my_x, peer_y, my_z),
            device_id_type=pl.DeviceIdType.MESH,
        )
        rdma.start()
        rdmas.append(rdma)
    for r in rdmas:
        r.wait_recv()

    g = gathT_ref[:, :]
    prev = None
    for i in range(K_OUT):
        cur = g if prev is None else jnp.where(g < prev, g, _NEG)
        m = jnp.max(cur, axis=0, keepdims=True)
        outT_ref[i:i + 1, :] = m
        prev = m
    out_ref[:, :] = outT_ref[:, :].T

    for r in rdmas:
        r.wait_send()


def kernel(x):
    cand = pl.pallas_call(
        _local_topk_body,
        grid=(M // ROW_BLK,),
        in_specs=[
            pl.BlockSpec((ROW_BLK, N_LOC), lambda i: (i, 0),
                         memory_space=pltpu.VMEM),
        ],
        out_specs=pl.BlockSpec((ROW_BLK, K_OUT), lambda i: (i, 0),
                               memory_space=pltpu.VMEM),
        out_shape=jax.ShapeDtypeStruct((M, K_OUT), jnp.float32),
    )(x)

    return pl.pallas_call(
        _merge_body,
        in_specs=[pl.BlockSpec(memory_space=pltpu.VMEM)],
        out_specs=pl.BlockSpec(memory_space=pltpu.VMEM),
        out_shape=jax.ShapeDtypeStruct((M, K_OUT), jnp.float32),
        scratch_shapes=[
            pltpu.VMEM((K_OUT, M), jnp.float32),
            pltpu.VMEM((N_Y * K_OUT, M), jnp.float32),
            pltpu.VMEM((K_OUT, M), jnp.float32),
            pltpu.SemaphoreType.DMA((N_Y - 1,)),
            pltpu.SemaphoreType.DMA((N_Y - 1,)),
        ],
        compiler_params=pltpu.CompilerParams(collective_id=0),
    )(cand)
